# baseline (speedup 1.0000x reference)
import jax
import jax.numpy as jnp
from jax.experimental import pallas as pl
from jax.experimental.pallas import tpu as pltpu

_W = 512


def _trim_body(x_hbm, v_ref, mask_ref, xo_hbm, vo_ref, mo_ref,
               zeros_buf, strad_buf, copy_sems, strad_rd_sem, strad_wr_sem):
    L = v_ref.shape[-1]
    nc = L // _W
    m = mask_ref[...]
    maxlen = jnp.maximum(jnp.max(jnp.sum(m, axis=-1)), 1)
    jlast = (maxlen - 1) // _W

    strad_rd = pltpu.make_async_copy(
        x_hbm.at[:, pl.ds(jlast * _W, _W)], strad_buf, strad_rd_sem)
    strad_rd.start()

    zeros_buf[...] = jnp.zeros_like(zeros_buf)

    for s in range(nc):
        @pl.when(s < jlast)
        def _live():
            pltpu.make_async_copy(
                x_hbm.at[:, pl.ds(s * _W, _W)],
                xo_hbm.at[:, pl.ds(s * _W, _W)],
                copy_sems.at[s]).start()

        @pl.when(s > jlast)
        def _dead():
            pltpu.make_async_copy(
                zeros_buf,
                xo_hbm.at[:, pl.ds(s * _W, _W)],
                copy_sems.at[s]).start()

    keep_row = jax.lax.broadcasted_iota(jnp.int32, (1, L), 1) < maxlen
    mo_ref[...] = jnp.logical_and(keep_row, m != 0)
    vo_ref[...] = jnp.where(keep_row, v_ref[...], 0.0)

    strad_rd.wait()
    col = jlast * _W + jax.lax.broadcasted_iota(jnp.int32, strad_buf.shape, 1)
    strad_buf[...] = jnp.where(col < maxlen, strad_buf[...], 0.0)
    strad_wr = pltpu.make_async_copy(
        strad_buf, xo_hbm.at[:, pl.ds(jlast * _W, _W)], strad_wr_sem)
    strad_wr.start()

    for s in range(nc):
        @pl.when(s < jlast)
        def _live_wait():
            pltpu.make_async_copy(
                x_hbm.at[:, pl.ds(s * _W, _W)],
                xo_hbm.at[:, pl.ds(s * _W, _W)],
                copy_sems.at[s]).wait()

        @pl.when(s > jlast)
        def _dead_wait():
            pltpu.make_async_copy(
                zeros_buf,
                xo_hbm.at[:, pl.ds(s * _W, _W)],
                copy_sems.at[s]).wait()

    strad_wr.wait()


def kernel(x, v, mask):
    B, C, L = x.shape
    Cv = v.shape[1]
    x2 = x.reshape(B * C, L)
    v2 = v.reshape(B * Cv, L)
    m2 = mask.reshape(B, L)

    x_out2, v_out2, m_out2 = pl.pallas_call(
        _trim_body,
        in_specs=[
            pl.BlockSpec(memory_space=pl.ANY),
            pl.BlockSpec((B * Cv, L), lambda: (0, 0)),
            pl.BlockSpec((B, L), lambda: (0, 0)),
        ],
        out_specs=[
            pl.BlockSpec(memory_space=pl.ANY),
            pl.BlockSpec((B * Cv, L), lambda: (0, 0)),
            pl.BlockSpec((B, L), lambda: (0, 0)),
        ],
        out_shape=[
            jax.ShapeDtypeStruct((B * C, L), x.dtype),
            jax.ShapeDtypeStruct((B * Cv, L), v.dtype),
            jax.ShapeDtypeStruct((B, L), jnp.bool_),
        ],
        scratch_shapes=[
            pltpu.VMEM((B * C, _W), x.dtype),
            pltpu.VMEM((B * C, _W), x.dtype),
            pltpu.SemaphoreType.DMA((L // _W,)),
            pltpu.SemaphoreType.DMA,
            pltpu.SemaphoreType.DMA,
        ],
    )(x2, v2, m2)

    return (
        x_out2.reshape(B, C, L),
        v_out2.reshape(B, Cv, L),
        m_out2.reshape(B, 1, L),
    )


# speedup vs baseline: 18.7074x; 18.7074x over previous
import jax
import jax.numpy as jnp
from jax.experimental import pallas as pl
from jax.experimental.pallas import tpu as pltpu

_ROWS = 512
_W = 512


def _trim_body(x_hbm, v_ref, mask_ref, xo_ref, vo_ref, mo_ref,
               maxlen_ref, strip_sems):
    i = pl.program_id(0)
    L = v_ref.shape[-1]
    nc = L // _W

    @pl.when(i == 0)
    def _prologue():
        m = mask_ref[...]
        maxlen = jnp.maximum(jnp.max(jnp.sum(m, axis=-1)), 1)
        maxlen_ref[0] = maxlen
        keep_row = jax.lax.broadcasted_iota(jnp.int32, (1, L), 1) < maxlen
        mo_ref[...] = jnp.logical_and(keep_row, m != 0)
        vo_ref[...] = jnp.where(keep_row, v_ref[...], 0.0)

    maxlen = maxlen_ref[0]
    jlast = (maxlen - 1) // _W

    for s in range(nc):
        @pl.when(s <= jlast)
        def _fetch():
            pltpu.make_async_copy(
                x_hbm.at[pl.ds(i * _ROWS, _ROWS), pl.ds(s * _W, _W)],
                xo_ref.at[:, pl.ds(s * _W, _W)],
                strip_sems.at[s]).start()

        @pl.when(s > jlast)
        def _zero():
            xo_ref[:, s * _W:(s + 1) * _W] = jnp.zeros((_ROWS, _W),
                                                       xo_ref.dtype)

    for s in range(nc):
        @pl.when(s <= jlast)
        def _wait():
            pltpu.make_async_copy(
                x_hbm.at[pl.ds(i * _ROWS, _ROWS), pl.ds(s * _W, _W)],
                xo_ref.at[:, pl.ds(s * _W, _W)],
                strip_sems.at[s]).wait()

    col = jlast * _W + jax.lax.broadcasted_iota(jnp.int32, (_ROWS, _W), 1)
    strad = xo_ref[:, pl.ds(jlast * _W, _W)]
    xo_ref[:, pl.ds(jlast * _W, _W)] = jnp.where(col < maxlen, strad, 0.0)


def kernel(x, v, mask):
    B, C, L = x.shape
    Cv = v.shape[1]
    x2 = x.reshape(B * C, L)
    v2 = v.reshape(B * Cv, L)
    m2 = mask.reshape(B, L)
    n_blocks = (B * C) // _ROWS

    x_out2, v_out2, m_out2 = pl.pallas_call(
        _trim_body,
        grid=(n_blocks,),
        in_specs=[
            pl.BlockSpec(memory_space=pl.ANY),
            pl.BlockSpec((B * Cv, L), lambda i: (0, 0)),
            pl.BlockSpec((B, L), lambda i: (0, 0)),
        ],
        out_specs=[
            pl.BlockSpec((_ROWS, L), lambda i: (i, 0)),
            pl.BlockSpec((B * Cv, L), lambda i: (0, 0)),
            pl.BlockSpec((B, L), lambda i: (0, 0)),
        ],
        out_shape=[
            jax.ShapeDtypeStruct((B * C, L), x.dtype),
            jax.ShapeDtypeStruct((B * Cv, L), v.dtype),
            jax.ShapeDtypeStruct((B, L), jnp.bool_),
        ],
        scratch_shapes=[
            pltpu.SMEM((1,), jnp.int32),
            pltpu.SemaphoreType.DMA((L // _W,)),
        ],
    )(x2, v2, m2)

    return (
        x_out2.reshape(B, C, L),
        v_out2.reshape(B, Cv, L),
        m_out2.reshape(B, 1, L),
    )
